# baseline (speedup 1.0000x reference)
import jax
import jax.numpy as jnp
from jax import lax
from jax.experimental import pallas as pl
from jax.experimental.pallas import tpu as pltpu
from jax.experimental.pallas import tpu_sc as plsc

N = 10000
E = 160000
D = 256
NC = 2
NS = 16
L = 16
NW = NC * NS
EPT = E // NW
CH = 64
EB = 8192
WSW = 128
DW = D + WSW


def _leaky(x):
    return jnp.where(x > 0, x, 0.3 * x)


def _mm1_body(x_ref, a_ref, o_ref):
    o_ref[...] = _leaky(
        jnp.dot(x_ref[...], a_ref[...], preferred_element_type=jnp.float32)
    )


def _stage1(h_src, Q):
    BR = 512
    return pl.pallas_call(
        _mm1_body,
        grid=(pl.cdiv(N, BR),),
        in_specs=[
            pl.BlockSpec((BR, D), lambda i: (i, 0)),
            pl.BlockSpec((D, D), lambda i: (0, 0)),
        ],
        out_specs=pl.BlockSpec((BR, D), lambda i: (i, 0)),
        out_shape=jax.ShapeDtypeStruct((N, D), jnp.float32),
    )(h_src, Q)


def _mm2_body(n_ref, ws_ref, h_ref, w1_ref, w2_ref, o_ref):
    r = n_ref[...] / jnp.maximum(ws_ref[...], 1.0)
    y = jnp.dot(r, w1_ref[...], preferred_element_type=jnp.float32)
    y = y + jnp.dot(h_ref[...], w2_ref[...], preferred_element_type=jnp.float32)
    o_ref[...] = _leaky(y)


def _stage3(n, ws256, h_dst, W1, W2):
    BR = 512
    return pl.pallas_call(
        _mm2_body,
        grid=(pl.cdiv(N, BR),),
        in_specs=[
            pl.BlockSpec((BR, D), lambda i: (i, 0)),
            pl.BlockSpec((BR, D), lambda i: (i, 0)),
            pl.BlockSpec((BR, D), lambda i: (i, 0)),
            pl.BlockSpec((D, D), lambda i: (0, 0)),
            pl.BlockSpec((D, D), lambda i: (0, 0)),
        ],
        out_specs=pl.BlockSpec((BR, D), lambda i: (i, 0)),
        out_shape=jax.ShapeDtypeStruct((N, D), jnp.float32),
    )(n, ws256, h_dst, W1, W2)


def _sc_body(nsrc, srce, we, m_out, ed_src, ed_w, grow, grow16, wrow, sem):
    cid = lax.axis_index("c")
    sid = lax.axis_index("s")
    wid = sid * NC + cid
    z116f = jnp.zeros((1, L), jnp.float32)
    ebase = wid * EPT

    def zpad(r, _):
        for c in range(1, WSW // L):
            grow[pl.ds(r, 1), pl.ds(D + c * L, L)] = z116f
        return 0
    lax.fori_loop(0, CH, zpad, 0)

    def zpad16(r, _):
        for c in range(1, WSW // L):
            grow16[pl.ds(r, 1), pl.ds(D + c * L, L)] = z116f
        return 0
    lax.fori_loop(0, L, zpad16, 0)

    ed_src[pl.ds(EPT - 8, L)] = jnp.zeros((L,), jnp.int32)
    ed_w[pl.ds(EPT - 8, L)] = jnp.zeros((L,), jnp.float32)
    pltpu.sync_copy(srce.at[pl.ds(ebase, EPT)], ed_src.at[pl.ds(0, EPT)])
    pltpu.sync_copy(we.at[pl.ds(ebase, EPT)], ed_w.at[pl.ds(0, EPT)])

    def do_block(goff, gbuf, rows):
        for j in range(rows // L):
            w16 = ed_w[pl.ds(goff + j * L, L)]
            for r in range(L):
                wrow[pl.ds(j * L + r, 1), pl.ds(0, L)] = z116f + w16[r]

        def scale(r2, _):
            wv = wrow[pl.ds(r2, 1), pl.ds(0, L)]
            for c in range(D // L):
                sl = (pl.ds(r2, 1), pl.ds(c * L, L))
                gbuf[sl] = gbuf[sl] * wv
            gbuf[pl.ds(r2, 1), pl.ds(D, L)] = wv
            return 0
        lax.fori_loop(0, rows, scale, 0)

    def chunk(i, _):
        goff = i * CH
        pltpu.async_copy(
            nsrc.at[ed_src.at[pl.ds(goff, CH)]], grow.at[:, pl.ds(0, D)], sem
        ).wait()
        do_block(goff, grow, CH)
        pltpu.sync_copy(grow, m_out.at[pl.ds(ebase + goff, CH)])
        return 0

    lax.fori_loop(0, EPT // CH, chunk, 0)

    toff = (EPT // CH) * CH
    pltpu.async_copy(
        nsrc.at[ed_src.at[pl.ds(toff, L)]], grow16.at[:, pl.ds(0, D)], sem
    ).wait()
    do_block(toff, grow16, L)
    pltpu.sync_copy(grow16.at[pl.ds(0, 8)], m_out.at[pl.ds(ebase + toff, 8)])


_sc_gather_scale = pl.kernel(
    _sc_body,
    out_type=jax.ShapeDtypeStruct((E, DW), jnp.float32),
    mesh=plsc.VectorSubcoreMesh(
        core_axis_name="c", subcore_axis_name="s", num_cores=NC, num_subcores=NS
    ),
    scratch_types=[
        pltpu.VMEM((EPT + L,), jnp.int32),
        pltpu.VMEM((EPT + L,), jnp.float32),
        pltpu.VMEM((CH, DW), jnp.float32),
        pltpu.VMEM((L, DW), jnp.float32),
        pltpu.VMEM((CH, L), jnp.float32),
        pltpu.SemaphoreType.DMA,
    ],
)


def _scatter_body(dst_ref, m_ref, acc_ref):
    pid = pl.program_id(0)

    @pl.when(pid == 0)
    def _init():
        acc_ref[...] = jnp.zeros_like(acc_ref)

    def body(e, _):
        idx = dst_ref[e]
        acc_ref[pl.ds(idx, 1), :] += m_ref[pl.ds(e, 1), :]
        return 0

    nfull = E // EB

    @pl.when(pid < nfull)
    def _full():
        lax.fori_loop(0, EB, body, 0, unroll=16)

    if E % EB:
        @pl.when(pid == nfull)
        def _tail():
            lax.fori_loop(0, E % EB, body, 0, unroll=16)


def _scatter_sum(dst, m):
    return pl.pallas_call(
        _scatter_body,
        grid=(pl.cdiv(E, EB),),
        in_specs=[
            pl.BlockSpec((EB,), lambda i: (i,), memory_space=pltpu.SMEM),
            pl.BlockSpec((EB, DW), lambda i: (i, 0)),
        ],
        out_specs=pl.BlockSpec((N, DW), lambda i: (0, 0)),
        out_shape=jax.ShapeDtypeStruct((N, DW), jnp.float32),
    )(dst, m)


def kernel(h_src, h_dst, edge_index, weights, Q, W):
    src = edge_index[0].astype(jnp.int32)
    dst = edge_index[1].astype(jnp.int32)
    w = weights.astype(jnp.float32)
    n_src = _stage1(h_src, Q)
    m = _sc_gather_scale(n_src, src, w)
    acc = _scatter_sum(dst, m)
    n = acc[:, :D]
    ws256 = jnp.tile(acc[:, D:D + 1], (1, D))
    z = _stage3(n, ws256, h_dst, W[:D], W[D:])
    return z


# speedup vs baseline: 1.0135x; 1.0135x over previous
import jax
import jax.numpy as jnp
from jax import lax
from jax.experimental import pallas as pl
from jax.experimental.pallas import tpu as pltpu
from jax.experimental.pallas import tpu_sc as plsc

N = 10000
E = 160000
D = 256
NC = 2
NS = 16
L = 16
NW = NC * NS
EPT = E // NW
CH = 64
EB = 4096
WSL = 128


def _leaky(x):
    return jnp.where(x > 0, x, 0.3 * x)


def _mm1_body(x_ref, a_ref, o_ref):
    o_ref[...] = _leaky(
        jnp.dot(x_ref[...], a_ref[...], preferred_element_type=jnp.float32)
    )


def _stage1(h_src, Q):
    BR = 512
    return pl.pallas_call(
        _mm1_body,
        grid=(pl.cdiv(N, BR),),
        in_specs=[
            pl.BlockSpec((BR, D), lambda i: (i, 0)),
            pl.BlockSpec((D, D), lambda i: (0, 0)),
        ],
        out_specs=pl.BlockSpec((BR, D), lambda i: (i, 0)),
        out_shape=jax.ShapeDtypeStruct((N, D), jnp.float32),
    )(h_src, Q)


def _mm2_body(n_ref, ws_ref, h_ref, w1_ref, w2_ref, o_ref):
    r = n_ref[...] / jnp.maximum(ws_ref[...], 1.0)
    y = jnp.dot(r, w1_ref[...], preferred_element_type=jnp.float32)
    y = y + jnp.dot(h_ref[...], w2_ref[...], preferred_element_type=jnp.float32)
    o_ref[...] = _leaky(y)


def _stage3(n, ws256, h_dst, W1, W2):
    BR = 512
    return pl.pallas_call(
        _mm2_body,
        grid=(pl.cdiv(N, BR),),
        in_specs=[
            pl.BlockSpec((BR, D), lambda i: (i, 0)),
            pl.BlockSpec((BR, D), lambda i: (i, 0)),
            pl.BlockSpec((BR, D), lambda i: (i, 0)),
            pl.BlockSpec((D, D), lambda i: (0, 0)),
            pl.BlockSpec((D, D), lambda i: (0, 0)),
        ],
        out_specs=pl.BlockSpec((BR, D), lambda i: (i, 0)),
        out_shape=jax.ShapeDtypeStruct((N, D), jnp.float32),
    )(n, ws256, h_dst, W1, W2)


def _sc_body(nsrc, srce, we, m_out, ed_src, ed_w, grow, grow16, wrow, sem):
    cid = lax.axis_index("c")
    sid = lax.axis_index("s")
    wid = sid * NC + cid
    z116f = jnp.zeros((1, L), jnp.float32)
    ebase = wid * EPT

    ed_src[pl.ds(EPT - 8, L)] = jnp.zeros((L,), jnp.int32)
    ed_w[pl.ds(EPT - 8, L)] = jnp.zeros((L,), jnp.float32)
    pltpu.sync_copy(srce.at[pl.ds(ebase, EPT)], ed_src.at[pl.ds(0, EPT)])
    pltpu.sync_copy(we.at[pl.ds(ebase, EPT)], ed_w.at[pl.ds(0, EPT)])

    def do_block(goff, gbuf, rows):
        for j in range(rows // L):
            w16 = ed_w[pl.ds(goff + j * L, L)]
            for r in range(L):
                wrow[pl.ds(j * L + r, 1), pl.ds(0, L)] = z116f + w16[r]

        def scale(r2, _):
            wv = wrow[pl.ds(r2, 1), pl.ds(0, L)]
            for c in range(D // L):
                sl = (pl.ds(r2, 1), pl.ds(c * L, L))
                gbuf[sl] = gbuf[sl] * wv
            return 0
        lax.fori_loop(0, rows, scale, 0)

    def chunk(i, _):
        goff = i * CH
        pltpu.async_copy(
            nsrc.at[ed_src.at[pl.ds(goff, CH)]], grow, sem
        ).wait()
        do_block(goff, grow, CH)
        pltpu.sync_copy(grow, m_out.at[pl.ds(ebase + goff, CH)])
        return 0

    lax.fori_loop(0, EPT // CH, chunk, 0)

    toff = (EPT // CH) * CH
    pltpu.async_copy(nsrc.at[ed_src.at[pl.ds(toff, L)]], grow16, sem).wait()
    do_block(toff, grow16, L)
    pltpu.sync_copy(grow16.at[pl.ds(0, 8)], m_out.at[pl.ds(ebase + toff, 8)])


_sc_gather_scale = pl.kernel(
    _sc_body,
    out_type=jax.ShapeDtypeStruct((E, D), jnp.float32),
    mesh=plsc.VectorSubcoreMesh(
        core_axis_name="c", subcore_axis_name="s", num_cores=NC, num_subcores=NS
    ),
    scratch_types=[
        pltpu.VMEM((EPT + L,), jnp.int32),
        pltpu.VMEM((EPT + L,), jnp.float32),
        pltpu.VMEM((CH, D), jnp.float32),
        pltpu.VMEM((L, D), jnp.float32),
        pltpu.VMEM((CH, L), jnp.float32),
        pltpu.SemaphoreType.DMA,
    ],
)


def _scatter_body(dst_ref, w_ref, m_ref, n_ref, ws_ref):
    pid = pl.program_id(0)

    @pl.when(pid == 0)
    def _init():
        n_ref[...] = jnp.zeros_like(n_ref)
        ws_ref[...] = jnp.zeros_like(ws_ref)

    def body(e, _):
        idx = dst_ref[e]
        wv = w_ref[e]
        n_ref[pl.ds(idx, 1), :] += m_ref[pl.ds(e, 1), :]
        ws_ref[pl.ds(idx, 1), :] += wv
        return 0

    nfull = E // EB

    @pl.when(pid < nfull)
    def _full():
        lax.fori_loop(0, EB, body, 0, unroll=8)

    if E % EB:
        @pl.when(pid == nfull)
        def _tail():
            lax.fori_loop(0, E % EB, body, 0, unroll=8)


def _scatter_sum(dst, w, m):
    return pl.pallas_call(
        _scatter_body,
        grid=(pl.cdiv(E, EB),),
        in_specs=[
            pl.BlockSpec((EB,), lambda i: (i,), memory_space=pltpu.SMEM),
            pl.BlockSpec((EB,), lambda i: (i,), memory_space=pltpu.SMEM),
            pl.BlockSpec((EB, D), lambda i: (i, 0)),
        ],
        out_specs=[
            pl.BlockSpec((N, D), lambda i: (0, 0)),
            pl.BlockSpec((N, WSL), lambda i: (0, 0)),
        ],
        out_shape=[
            jax.ShapeDtypeStruct((N, D), jnp.float32),
            jax.ShapeDtypeStruct((N, WSL), jnp.float32),
        ],
    )(dst, w, m)


def kernel(h_src, h_dst, edge_index, weights, Q, W):
    src = edge_index[0].astype(jnp.int32)
    dst = edge_index[1].astype(jnp.int32)
    w = weights.astype(jnp.float32)
    n_src = _stage1(h_src, Q)
    m = _sc_gather_scale(n_src, src, w)
    n, ws = _scatter_sum(dst, w, m)
    ws256 = jnp.tile(ws[:, :1], (1, D))
    z = _stage3(n, ws256, h_dst, W[:D], W[D:])
    return z


# speedup vs baseline: 1.1219x; 1.1070x over previous
import jax
import jax.numpy as jnp
from jax import lax
from jax.experimental import pallas as pl
from jax.experimental.pallas import tpu as pltpu
from jax.experimental.pallas import tpu_sc as plsc

N = 10000
E = 160000
D = 256
NC = 2
NS = 16
L = 16
NW = NC * NS
EPT = E // NW
CH = 64
EB = 4096
WSL = 128


def _leaky(x):
    return jnp.where(x > 0, x, 0.3 * x)


def _mm1_body(x_ref, a_ref, o_ref):
    o_ref[...] = _leaky(
        jnp.dot(x_ref[...], a_ref[...], preferred_element_type=jnp.float32)
    )


def _stage1(h_src, Q):
    BR = 512
    return pl.pallas_call(
        _mm1_body,
        grid=(pl.cdiv(N, BR),),
        in_specs=[
            pl.BlockSpec((BR, D), lambda i: (i, 0)),
            pl.BlockSpec((D, D), lambda i: (0, 0)),
        ],
        out_specs=pl.BlockSpec((BR, D), lambda i: (i, 0)),
        out_shape=jax.ShapeDtypeStruct((N, D), jnp.float32),
    )(h_src, Q)


def _mm2_body(n_ref, ws_ref, h_ref, w1_ref, w2_ref, o_ref):
    r = n_ref[...] / jnp.maximum(ws_ref[...], 1.0)
    y = jnp.dot(r, w1_ref[...], preferred_element_type=jnp.float32)
    y = y + jnp.dot(h_ref[...], w2_ref[...], preferred_element_type=jnp.float32)
    o_ref[...] = _leaky(y)


def _stage3(n, ws256, h_dst, W1, W2):
    BR = 512
    return pl.pallas_call(
        _mm2_body,
        grid=(pl.cdiv(N, BR),),
        in_specs=[
            pl.BlockSpec((BR, D), lambda i: (i, 0)),
            pl.BlockSpec((BR, D), lambda i: (i, 0)),
            pl.BlockSpec((BR, D), lambda i: (i, 0)),
            pl.BlockSpec((D, D), lambda i: (0, 0)),
            pl.BlockSpec((D, D), lambda i: (0, 0)),
        ],
        out_specs=pl.BlockSpec((BR, D), lambda i: (i, 0)),
        out_shape=jax.ShapeDtypeStruct((N, D), jnp.float32),
    )(n, ws256, h_dst, W1, W2)


def _sc_body(nsrc, srce, we, m_out,
             ed_src, ed_w, grow, growb, grow16, wrow, sem, semb):
    cid = lax.axis_index("c")
    sid = lax.axis_index("s")
    wid = sid * NC + cid
    z116f = jnp.zeros((1, L), jnp.float32)
    ebase = wid * EPT

    ed_src[pl.ds(EPT - 8, L)] = jnp.zeros((L,), jnp.int32)
    ed_w[pl.ds(EPT - 8, L)] = jnp.zeros((L,), jnp.float32)
    pltpu.sync_copy(srce.at[pl.ds(ebase, EPT)], ed_src.at[pl.ds(0, EPT)])
    pltpu.sync_copy(we.at[pl.ds(ebase, EPT)], ed_w.at[pl.ds(0, EPT)])

    def do_block(goff, gbuf, rows):
        for j in range(rows // L):
            w16 = ed_w[pl.ds(goff + j * L, L)]
            for r in range(L):
                wrow[pl.ds(j * L + r, 1), pl.ds(0, L)] = z116f + w16[r]

        def scale(r2, _):
            wv = wrow[pl.ds(r2, 1), pl.ds(0, L)]
            for c in range(D // L):
                sl = (pl.ds(r2, 1), pl.ds(c * L, L))
                gbuf[sl] = gbuf[sl] * wv
            return 0
        lax.fori_loop(0, rows, scale, 0)

    nch = EPT // CH

    def issue(i, gbuf, s):
        pltpu.async_copy(nsrc.at[ed_src.at[pl.ds(i * CH, CH)]], gbuf, s)

    issue(0, grow, sem)
    issue(1, growb, semb)

    def consume(i, gbuf, s):
        pltpu.make_async_copy(
            nsrc.at[ed_src.at[pl.ds(i * CH, CH)]], gbuf, s
        ).wait()
        do_block(i * CH, gbuf, CH)
        pltpu.sync_copy(gbuf, m_out.at[pl.ds(ebase + i * CH, CH)])

    def chunk2(i2, _):
        ia = 2 * i2
        consume(ia, grow, sem)

        @pl.when(ia + 2 < nch)
        def _pf_a():
            issue(ia + 2, grow, sem)

        consume(ia + 1, growb, semb)

        @pl.when(ia + 3 < nch)
        def _pf_b():
            issue(ia + 3, growb, semb)
        return 0

    lax.fori_loop(0, nch // 2, chunk2, 0)

    toff = (EPT // CH) * CH
    pltpu.async_copy(nsrc.at[ed_src.at[pl.ds(toff, L)]], grow16, sem).wait()
    do_block(toff, grow16, L)
    pltpu.sync_copy(grow16.at[pl.ds(0, 8)], m_out.at[pl.ds(ebase + toff, 8)])


_sc_gather_scale = pl.kernel(
    _sc_body,
    out_type=jax.ShapeDtypeStruct((E, D), jnp.float32),
    mesh=plsc.VectorSubcoreMesh(
        core_axis_name="c", subcore_axis_name="s", num_cores=NC, num_subcores=NS
    ),
    scratch_types=[
        pltpu.VMEM((EPT + L,), jnp.int32),
        pltpu.VMEM((EPT + L,), jnp.float32),
        pltpu.VMEM((CH, D), jnp.float32),
        pltpu.VMEM((CH, D), jnp.float32),
        pltpu.VMEM((L, D), jnp.float32),
        pltpu.VMEM((CH, L), jnp.float32),
        pltpu.SemaphoreType.DMA,
        pltpu.SemaphoreType.DMA,
    ],
)


def _scatter_body(dst_ref, w_ref, m_ref, n_ref, ws_ref):
    pid = pl.program_id(0)

    @pl.when(pid == 0)
    def _init():
        n_ref[...] = jnp.zeros_like(n_ref)
        ws_ref[...] = jnp.zeros_like(ws_ref)

    def body(e, _):
        idx = dst_ref[e]
        wv = w_ref[e]
        n_ref[pl.ds(idx, 1), :] += m_ref[pl.ds(e, 1), :]
        ws_ref[pl.ds(idx, 1), :] += wv
        return 0

    nfull = E // EB

    @pl.when(pid < nfull)
    def _full():
        lax.fori_loop(0, EB, body, 0, unroll=8)

    if E % EB:
        @pl.when(pid == nfull)
        def _tail():
            lax.fori_loop(0, E % EB, body, 0, unroll=8)


def _scatter_sum(dst, w, m):
    return pl.pallas_call(
        _scatter_body,
        grid=(pl.cdiv(E, EB),),
        in_specs=[
            pl.BlockSpec((EB,), lambda i: (i,), memory_space=pltpu.SMEM),
            pl.BlockSpec((EB,), lambda i: (i,), memory_space=pltpu.SMEM),
            pl.BlockSpec((EB, D), lambda i: (i, 0)),
        ],
        out_specs=[
            pl.BlockSpec((N, D), lambda i: (0, 0)),
            pl.BlockSpec((N, WSL), lambda i: (0, 0)),
        ],
        out_shape=[
            jax.ShapeDtypeStruct((N, D), jnp.float32),
            jax.ShapeDtypeStruct((N, WSL), jnp.float32),
        ],
    )(dst, w, m)


def kernel(h_src, h_dst, edge_index, weights, Q, W):
    src = edge_index[0].astype(jnp.int32)
    dst = edge_index[1].astype(jnp.int32)
    w = weights.astype(jnp.float32)
    n_src = _stage1(h_src, Q)
    m = _sc_gather_scale(n_src, src, w)
    n, ws = _scatter_sum(dst, w, m)
    ws256 = jnp.tile(ws[:, :1], (1, D))
    z = _stage3(n, ws256, h_dst, W[:D], W[D:])
    return z
